# baseline (speedup 1.0000x reference)
import jax
import jax.numpy as jnp
import numpy as np
from jax.experimental import pallas as pl
from jax.experimental.pallas import tpu as pltpu

_T = 512
_NXY = 48
_S = _NXY * _NXY
_K = 4 * 5000
_H1, _H2 = 64, 128
_EL_NORM = 2.5066
_GAUSS_NORM = 0.3989422804
_BIN_SIGMA = 5.0

_CHUNK = 2048
_KPAD = 20480
_SBLK = 2304
_IBLK = _SBLK // _NXY


def _body(consts_ref, data_ref, zcol_ref, gxc_ref, gyc_ref, w1t_ref, b1c_ref,
          w2t_ref, b2c_ref, w3t_ref, out_ref):
    k = pl.program_id(1)

    xy_t = data_ref[0:2, :]
    m_t = data_ref[2:3, :]

    h = jnp.dot(w1t_ref[...], xy_t, preferred_element_type=jnp.float32)
    h = jnp.maximum(h + b1c_ref[...], 0.0)
    h = jnp.dot(w2t_ref[...], h, preferred_element_type=jnp.float32)
    h = jnp.maximum(h + b2c_ref[...], 0.0)
    resp_t = jnp.dot(w3t_ref[...], h, preferred_element_type=jnp.float32)
    coef_t = (resp_t + consts_ref[2]) * m_t * consts_ref[0]

    x_t = data_ref[0:1, :]
    y_t = data_ref[1:2, :]
    dx = gxc_ref[...] - x_t
    dy = gyc_ref[...] - y_t
    ex = jnp.exp(dx * dx * consts_ref[1]) * coef_t
    ey = jnp.exp(dy * dy * consts_ref[1])
    exb = ex.astype(jnp.bfloat16)
    eyb = ey.astype(jnp.bfloat16)
    sp_t = (exb[:, None, :] * eyb[None, :, :]).reshape(_SBLK, _CHUNK)

    z = zcol_ref[...]
    t = jax.lax.broadcasted_iota(jnp.int32, (1, _T), 1).astype(jnp.float32)
    dt = t - z
    ev = jnp.exp(dt * dt * (-1.0 / (2.0 * _BIN_SIGMA)))
    evb = ev.astype(jnp.bfloat16)

    @pl.when(k == 0)
    def _():
        out_ref[...] = jnp.zeros_like(out_ref)

    out_ref[...] = out_ref[...] + jnp.dot(
        sp_t, evb, preferred_element_type=jnp.float32)


def kernel(simulator_input, z_positions, mask, W1, b1, W2, b2, W3, b3,
           el_spread, sensor_locations):
    f32 = jnp.float32
    xy = simulator_input.reshape(_K, 2).astype(f32)
    pad = _KPAD - _K
    data = jnp.concatenate([xy.T, mask.reshape(1, _K).astype(f32)], axis=0)
    data = jnp.pad(data, ((0, 0), (0, pad)))
    zcol = jnp.pad(z_positions.reshape(_K, 1).astype(f32), ((0, pad), (0, 0)))

    gxy = sensor_locations.reshape(_S, 2)
    gxc = gxy[:: _NXY, 0:1]
    gyc = gxy[: _NXY, 1:2]

    es = el_spread[0].astype(f32)
    coef_scale = (100.0 / (es * _EL_NORM)) * (_GAUSS_NORM / np.sqrt(_BIN_SIGMA))
    inv_2es2 = -0.5 / (es * es)
    consts = jnp.stack([coef_scale, inv_2es2, b3[0].astype(f32)])

    grid = (_S // _SBLK, _KPAD // _CHUNK)

    out = pl.pallas_call(
        _body,
        grid=grid,
        in_specs=[
            pl.BlockSpec(memory_space=pltpu.SMEM),
            pl.BlockSpec((3, _CHUNK), lambda s, k: (0, k)),
            pl.BlockSpec((_CHUNK, 1), lambda s, k: (k, 0)),
            pl.BlockSpec((_IBLK, 1), lambda s, k: (s, 0)),
            pl.BlockSpec((_NXY, 1), lambda s, k: (0, 0)),
            pl.BlockSpec((_H1, 2), lambda s, k: (0, 0)),
            pl.BlockSpec((_H1, 1), lambda s, k: (0, 0)),
            pl.BlockSpec((_H2, _H1), lambda s, k: (0, 0)),
            pl.BlockSpec((_H2, 1), lambda s, k: (0, 0)),
            pl.BlockSpec((1, _H2), lambda s, k: (0, 0)),
        ],
        out_specs=pl.BlockSpec((_SBLK, _T), lambda s, k: (s, 0)),
        out_shape=jax.ShapeDtypeStruct((_S, _T), f32),
        compiler_params=pltpu.CompilerParams(
            dimension_semantics=("arbitrary", "arbitrary"),
            vmem_limit_bytes=56 * 1024 * 1024,
        ),
    )(consts, data, zcol, gxc, gyc,
      W1.T.astype(f32), b1.reshape(_H1, 1).astype(f32),
      W2.T.astype(f32), b2.reshape(_H2, 1).astype(f32),
      W3.T.astype(f32))

    return out.reshape(_NXY, _NXY, _T)


# speedup vs baseline: 1.1436x; 1.1436x over previous
import jax
import jax.numpy as jnp
import numpy as np
from jax.experimental import pallas as pl
from jax.experimental.pallas import tpu as pltpu

_T = 512
_NXY = 48
_S = _NXY * _NXY
_K = 4 * 5000
_H1, _H2 = 64, 128
_EL_NORM = 2.5066
_GAUSS_NORM = 0.3989422804
_BIN_SIGMA = 5.0

_CHUNK = 4096
_KPAD = 20480
_SBLK = 2304
_IBLK = _SBLK // _NXY


def _body(consts_ref, data_ref, zcol_ref, gxc_ref, gyc_ref, w1t_ref, b1c_ref,
          w2t_ref, b2c_ref, w3t_ref, out_ref):
    k = pl.program_id(1)

    xy_t = data_ref[0:2, :]
    m_t = data_ref[2:3, :]

    h = jnp.dot(w1t_ref[...], xy_t, preferred_element_type=jnp.float32)
    h = jnp.maximum(h + b1c_ref[...], 0.0)
    h = jnp.dot(w2t_ref[...], h, preferred_element_type=jnp.float32)
    h = jnp.maximum(h + b2c_ref[...], 0.0)
    resp_t = jnp.dot(w3t_ref[...], h, preferred_element_type=jnp.float32)
    coef_t = (resp_t + consts_ref[2]) * m_t * consts_ref[0]

    x_t = data_ref[0:1, :]
    y_t = data_ref[1:2, :]
    dx = gxc_ref[...] - x_t
    dy = gyc_ref[...] - y_t
    ex = jnp.exp(dx * dx * consts_ref[1]) * coef_t
    ey = jnp.exp(dy * dy * consts_ref[1])
    exb = ex.astype(jnp.bfloat16)
    eyb = ey.astype(jnp.bfloat16)
    sp_t = (exb[:, None, :] * eyb[None, :, :]).reshape(_SBLK, _CHUNK)

    z = zcol_ref[...]
    t = jax.lax.broadcasted_iota(jnp.int32, (1, _T), 1).astype(jnp.float32)
    dt = t - z
    ev = jnp.exp(dt * dt * (-1.0 / (2.0 * _BIN_SIGMA)))
    evb = ev.astype(jnp.bfloat16)

    acc = jnp.dot(sp_t, evb, preferred_element_type=jnp.float32)

    @pl.when(k == 0)
    def _():
        out_ref[...] = acc

    @pl.when(k != 0)
    def _():
        out_ref[...] = out_ref[...] + acc


def kernel(simulator_input, z_positions, mask, W1, b1, W2, b2, W3, b3,
           el_spread, sensor_locations):
    f32 = jnp.float32
    xy = simulator_input.reshape(_K, 2).astype(f32)
    pad = _KPAD - _K
    data = jnp.concatenate([xy.T, mask.reshape(1, _K).astype(f32)], axis=0)
    data = jnp.pad(data, ((0, 0), (0, pad)))
    zcol = jnp.pad(z_positions.reshape(_K, 1).astype(f32), ((0, pad), (0, 0)))

    gxy = sensor_locations.reshape(_S, 2)
    gxc = gxy[:: _NXY, 0:1]
    gyc = gxy[: _NXY, 1:2]

    es = el_spread[0].astype(f32)
    coef_scale = (100.0 / (es * _EL_NORM)) * (_GAUSS_NORM / np.sqrt(_BIN_SIGMA))
    inv_2es2 = -0.5 / (es * es)
    consts = jnp.stack([coef_scale, inv_2es2, b3[0].astype(f32)])

    grid = (_S // _SBLK, _KPAD // _CHUNK)

    out = pl.pallas_call(
        _body,
        grid=grid,
        in_specs=[
            pl.BlockSpec(memory_space=pltpu.SMEM),
            pl.BlockSpec((3, _CHUNK), lambda s, k: (0, k)),
            pl.BlockSpec((_CHUNK, 1), lambda s, k: (k, 0)),
            pl.BlockSpec((_IBLK, 1), lambda s, k: (s, 0)),
            pl.BlockSpec((_NXY, 1), lambda s, k: (0, 0)),
            pl.BlockSpec((_H1, 2), lambda s, k: (0, 0)),
            pl.BlockSpec((_H1, 1), lambda s, k: (0, 0)),
            pl.BlockSpec((_H2, _H1), lambda s, k: (0, 0)),
            pl.BlockSpec((_H2, 1), lambda s, k: (0, 0)),
            pl.BlockSpec((1, _H2), lambda s, k: (0, 0)),
        ],
        out_specs=pl.BlockSpec((_SBLK, _T), lambda s, k: (s, 0)),
        out_shape=jax.ShapeDtypeStruct((_S, _T), f32),
        compiler_params=pltpu.CompilerParams(
            dimension_semantics=("arbitrary", "arbitrary"),
            vmem_limit_bytes=56 * 1024 * 1024,
        ),
    )(consts, data, zcol, gxc, gyc,
      W1.T.astype(f32), b1.reshape(_H1, 1).astype(f32),
      W2.T.astype(f32), b2.reshape(_H2, 1).astype(f32),
      W3.T.astype(f32))

    return out.reshape(_NXY, _NXY, _T)
